# Initial kernel scaffold; baseline (speedup 1.0000x reference)
import jax
import jax.numpy as jnp
from jax.experimental import pallas as pl


def kernel(x, edge_index, edge_attr, batch, Wl, Wr, We, att, bias, Wc, bc):
    raise NotImplementedError("write your pallas kernel here")


# speedup vs baseline: 4.3322x; 4.3322x over previous
import functools

import jax
import jax.numpy as jnp
from jax import lax
from jax.experimental import pallas as pl
from jax.experimental.pallas import tpu as pltpu
from jax.experimental.pallas import tpu_sc as plsc

NC = 2
NS = 16
NW = NC * NS
LANES = 16
CHUNK = 128


def _cdiv(a, b):
    return (a + b - 1) // b


_DEBUG_JNP_TAIL = False
_DEBUG_JNP_EPILOGUE = False


def _k1_body(e_total, n_blocks, ea_ref, we_ref, el_ref, cs_ref, sl_ref):
    i = pl.program_id(0)

    @pl.when(i == 0)
    def _():
        cs_ref[...] = jnp.zeros_like(cs_ref)

    ea = ea_ref[...]
    el_ref[...] = jnp.dot(ea, we_ref[...], preferred_element_type=jnp.float32)
    cs_ref[0, :16] = cs_ref[0, :16] + jnp.sum(ea, axis=0)

    @pl.when(i == n_blocks - 1)
    def _():
        mean = (cs_ref[0, :16] / jnp.float32(e_total)).reshape(1, 16)
        sl_ref[...] = jnp.dot(mean, we_ref[...], preferred_element_type=jnp.float32)


def _edge_transform(ea_pad, We, e_total):
    e_pad = ea_pad.shape[0]
    blk = 4096
    n_blocks = e_pad // blk
    return pl.pallas_call(
        functools.partial(_k1_body, e_total, n_blocks),
        grid=(n_blocks,),
        in_specs=[
            pl.BlockSpec((blk, 16), lambda i: (i, 0)),
            pl.BlockSpec((16, 128), lambda i: (0, 0)),
        ],
        out_specs=[
            pl.BlockSpec((blk, 128), lambda i: (i, 0)),
            pl.BlockSpec((8, 128), lambda i: (0, 0)),
            pl.BlockSpec((1, 128), lambda i: (0, 0)),
        ],
        out_shape=[
            jax.ShapeDtypeStruct((e_pad, 128), jnp.float32),
            jax.ShapeDtypeStruct((8, 128), jnp.float32),
            jax.ShapeDtypeStruct((1, 128), jnp.float32),
        ],
        compiler_params=pltpu.CompilerParams(
            dimension_semantics=("arbitrary",),
        ),
    )(ea_pad, We)


def _k2_body(x_ref, wl_ref, wr_ref, sl_ref, att_ref, xl_ref, xr_ref, as_ref):
    xb = x_ref[...]
    xl = jnp.dot(xb, wl_ref[...], preferred_element_type=jnp.float32)
    xr = jnp.dot(xb, wr_ref[...], preferred_element_type=jnp.float32)
    xl_ref[...] = xl
    xr_ref[...] = xr
    t = xl + xr + sl_ref[...]
    t = jnp.maximum(t, 0.2 * t)
    as_ref[...] = jnp.dot(t, att_ref[...], preferred_element_type=jnp.float32)


def _node_transform(x_pad, Wl, Wr, sl, att_col):
    n_pad = x_pad.shape[0]
    blk = 1280
    n_blocks = n_pad // blk
    return pl.pallas_call(
        _k2_body,
        grid=(n_blocks,),
        in_specs=[
            pl.BlockSpec((blk, 128), lambda i: (i, 0)),
            pl.BlockSpec((128, 128), lambda i: (0, 0)),
            pl.BlockSpec((128, 128), lambda i: (0, 0)),
            pl.BlockSpec((1, 128), lambda i: (0, 0)),
            pl.BlockSpec((128, 1), lambda i: (0, 0)),
        ],
        out_specs=[
            pl.BlockSpec((blk, 128), lambda i: (i, 0)),
            pl.BlockSpec((blk, 128), lambda i: (i, 0)),
            pl.BlockSpec((blk, 1), lambda i: (i, 0)),
        ],
        out_shape=[
            jax.ShapeDtypeStruct((n_pad, 128), jnp.float32),
            jax.ShapeDtypeStruct((n_pad, 128), jnp.float32),
            jax.ShapeDtypeStruct((n_pad, 1), jnp.float32),
        ],
    )(x_pad, Wl, Wr, sl, att_col)


def _butterfly_sum(v, perms):
    for p in perms:
        v = v + v.at[p].get(mode="promise_in_bounds")
    return v


def _k3_body(epw, xl_hbm, xr_hbm, el_hbm, src_hbm, dst_hbm, att_hbm,
             araw_hbm, tmax_hbm,
             s_idx, d_idx, xlb, xrb, elb, ab, attv, mb,
             sem0, sem1, sem2):
    wid = lax.axis_index("s") * NC + lax.axis_index("c")
    base = wid * epw
    n_chunks = epw // CHUNK

    pltpu.sync_copy(att_hbm, attv)
    att_g = [attv[pl.ds(g * 16, 16)] for g in range(8)]
    lane = lax.iota(jnp.int32, 16)
    perms = [lane ^ (1 << k) for k in range(4)]

    def chunk_body(c, rmax):
        off = base + c * CHUNK
        pltpu.sync_copy(src_hbm.at[pl.ds(off, CHUNK)], s_idx)
        pltpu.sync_copy(dst_hbm.at[pl.ds(off, CHUNK)], d_idx)
        cp0 = pltpu.async_copy(xl_hbm.at[s_idx], xlb, sem0)
        cp1 = pltpu.async_copy(xr_hbm.at[d_idx], xrb, sem1)
        cp2 = pltpu.async_copy(el_hbm.at[pl.ds(off, CHUNK)], elb, sem2)
        cp0.wait()
        cp1.wait()
        cp2.wait()

        def grp_body(j16, rmax_in):
            av = jnp.zeros((16,), jnp.float32)
            for jj in range(16):
                j = j16 * 16 + jj
                acc = jnp.zeros((16,), jnp.float32)
                for g in range(8):
                    v = (xlb[j, pl.ds(g * 16, 16)]
                         + xrb[j, pl.ds(g * 16, 16)]
                         + elb[j, pl.ds(g * 16, 16)])
                    v = jnp.maximum(v, 0.2 * v)
                    acc = acc + v * att_g[g]
                s = _butterfly_sum(acc, perms)
                av = jnp.where(lane == jj, s, av)
            ab[pl.ds(j16 * 16, 16)] = av
            return jnp.maximum(rmax_in, av)

        rmax = lax.fori_loop(0, CHUNK // 16, grp_body, rmax)
        pltpu.sync_copy(ab, araw_hbm.at[pl.ds(off, CHUNK)])
        return rmax

    rmax = lax.fori_loop(0, n_chunks,
                         chunk_body,
                         jnp.full((16,), -jnp.inf, jnp.float32))
    mb[...] = rmax
    pltpu.sync_copy(mb, tmax_hbm.at[wid])


def _edge_logits(xl, xr, el, src, dst, att):
    e_pad = src.shape[0]
    epw = e_pad // NW
    mesh = plsc.VectorSubcoreMesh(core_axis_name="c", subcore_axis_name="s",
                                  num_cores=NC, num_subcores=NS)
    kfun = pl.kernel(
        functools.partial(_k3_body, epw),
        out_type=[
            jax.ShapeDtypeStruct((e_pad,), jnp.float32),
            jax.ShapeDtypeStruct((NW, LANES), jnp.float32),
        ],
        mesh=mesh,
        scratch_types=[
            pltpu.VMEM((CHUNK,), jnp.int32),
            pltpu.VMEM((CHUNK,), jnp.int32),
            pltpu.VMEM((CHUNK, 128), jnp.float32),
            pltpu.VMEM((CHUNK, 128), jnp.float32),
            pltpu.VMEM((CHUNK, 128), jnp.float32),
            pltpu.VMEM((CHUNK,), jnp.float32),
            pltpu.VMEM((128,), jnp.float32),
            pltpu.VMEM((LANES,), jnp.float32),
            pltpu.SemaphoreType.DMA,
            pltpu.SemaphoreType.DMA,
            pltpu.SemaphoreType.DMA,
        ],
    )
    return kfun(xl, xr, el, src, dst, att)


def _k4_body(epw, n_acc, chb, xl_hbm, src_hbm, dst_hbm, araw_hbm, m_hbm,
             pn_hbm,
             s_idx, d_idx, xlb, ab, rowb, mv,
             shn, sem0):
    cid = lax.axis_index("c")
    sid = lax.axis_index("s")
    wid = sid * NC + cid
    base = wid * epw
    n_chunks = epw // chb
    rows_per_tile = n_acc // NS

    lane = lax.iota(jnp.int32, 16)
    zero16 = jnp.zeros((16,), jnp.float32)

    def zrow(r, _):
        for g in range(8):
            rowb[r, pl.ds(g * 16, 16)] = zero16
        return 0

    lax.fori_loop(0, chb, zrow, 0)

    def zstripe(z, _):
        pltpu.sync_copy(rowb, shn.at[pl.ds(sid * rows_per_tile + z * chb, chb)])
        return 0

    lax.fori_loop(0, rows_per_tile // chb, zstripe, 0)
    plsc.subcore_barrier()

    pltpu.sync_copy(m_hbm, mv)
    mvec = mv[...]

    def chunk_body(c, _):
        off = base + c * chb
        pltpu.sync_copy(src_hbm.at[pl.ds(off, chb)], s_idx)
        pltpu.sync_copy(dst_hbm.at[pl.ds(off, chb)], d_idx)
        cp0 = pltpu.async_copy(xl_hbm.at[s_idx], xlb, sem0)
        pltpu.sync_copy(araw_hbm.at[pl.ds(off, chb)], ab)
        cp0.wait()

        def grp_body(j16, _):
            av = ab[pl.ds(j16 * 16, 16)]
            wv = jnp.exp(av - mvec)
            for jj in range(16):
                j = j16 * 16 + jj
                wb = wv.at[lane * 0 + jj].get(mode="promise_in_bounds")
                for g in range(8):
                    rowb[j, pl.ds(g * 16, 16)] = xlb[j, pl.ds(g * 16, 16)] * wb
            return 0

        lax.fori_loop(0, chb // 16, grp_body, 0)
        pltpu.sync_copy(rowb, shn.at[d_idx], add=True)
        return 0

    lax.fori_loop(0, n_chunks, chunk_body, 0)
    plsc.subcore_barrier()

    r0 = sid * rows_per_tile
    pltpu.sync_copy(shn.at[pl.ds(r0, rows_per_tile)],
                    pn_hbm.at[pl.ds(cid * n_acc + r0, rows_per_tile)])


def _scatter_messages(xl, src, dst, araw, m_vec, n_acc):
    e_pad = src.shape[0]
    epw = e_pad // NW
    chb = 64
    mesh = plsc.VectorSubcoreMesh(core_axis_name="c", subcore_axis_name="s",
                                  num_cores=NC, num_subcores=NS)
    kfun = pl.kernel(
        functools.partial(_k4_body, epw, n_acc, chb),
        out_type=jax.ShapeDtypeStruct((NC * n_acc, 128), jnp.float32),
        mesh=mesh,
        scratch_types=[
            pltpu.VMEM((chb,), jnp.int32),
            pltpu.VMEM((chb,), jnp.int32),
            pltpu.VMEM((chb, 128), jnp.float32),
            pltpu.VMEM((chb,), jnp.float32),
            pltpu.VMEM((chb, 128), jnp.float32),
            pltpu.VMEM((LANES,), jnp.float32),
            pltpu.VMEM_SHARED((n_acc, 128), jnp.float32),
            pltpu.SemaphoreType.DMA,
        ],
    )
    return kfun(xl, src, dst, araw, m_vec)


def _k4b_body(epw, n_acc, chb, dst_hbm, araw_hbm, m_hbm,
              pd_hbm,
              d_idx, ab, wrowb, mv, shd):
    cid = lax.axis_index("c")
    sid = lax.axis_index("s")
    wid = sid * NC + cid
    base = wid * epw
    n_chunks = epw // chb
    rows_per_tile = n_acc // NS

    lane = lax.iota(jnp.int32, 16)
    zero16 = jnp.zeros((16,), jnp.float32)

    def zrow(r, _):
        for g in range(8):
            wrowb[r, pl.ds(g * 16, 16)] = zero16
        return 0

    lax.fori_loop(0, chb, zrow, 0)

    def zstripe(z, _):
        pltpu.sync_copy(wrowb, shd.at[pl.ds(sid * rows_per_tile + z * chb, chb)])
        return 0

    lax.fori_loop(0, rows_per_tile // chb, zstripe, 0)
    plsc.subcore_barrier()

    pltpu.sync_copy(m_hbm, mv)
    mvec = mv[...]

    def chunk_body(c, _):
        off = base + c * chb
        pltpu.sync_copy(dst_hbm.at[pl.ds(off, chb)], d_idx)
        pltpu.sync_copy(araw_hbm.at[pl.ds(off, chb)], ab)

        def grp_body(j16, _):
            av = ab[pl.ds(j16 * 16, 16)]
            wv = jnp.exp(av - mvec)
            for jj in range(16):
                j = j16 * 16 + jj
                wb = wv.at[lane * 0 + jj].get(mode="promise_in_bounds")
                wrowb[j, pl.ds(0, 16)] = jnp.where(lane == 0, wb, zero16)
            return 0

        lax.fori_loop(0, chb // 16, grp_body, 0)
        pltpu.sync_copy(wrowb, shd.at[d_idx], add=True)
        return 0

    lax.fori_loop(0, n_chunks, chunk_body, 0)
    plsc.subcore_barrier()

    r0 = sid * rows_per_tile
    pltpu.sync_copy(shd.at[pl.ds(r0, rows_per_tile)],
                    pd_hbm.at[pl.ds(cid * n_acc + r0, rows_per_tile)])


def _scatter_weights(dst, araw, m_vec, n_acc):
    e_pad = dst.shape[0]
    epw = e_pad // NW
    chb = 64
    mesh = plsc.VectorSubcoreMesh(core_axis_name="c", subcore_axis_name="s",
                                  num_cores=NC, num_subcores=NS)
    kfun = pl.kernel(
        functools.partial(_k4b_body, epw, n_acc, chb),
        out_type=jax.ShapeDtypeStruct((NC * n_acc, 128), jnp.float32),
        mesh=mesh,
        scratch_types=[
            pltpu.VMEM((chb,), jnp.int32),
            pltpu.VMEM((chb,), jnp.float32),
            pltpu.VMEM((chb, 128), jnp.float32),
            pltpu.VMEM((LANES,), jnp.float32),
            pltpu.VMEM_SHARED((n_acc, 128), jnp.float32),
        ],
    )
    return kfun(dst, araw, m_vec)


def _k5_body(n_blocks, pn_ref, pd_ref, as_ref, xl_ref, oh_ref, bias_ref,
             m_ref, wc_ref, bc_ref, out_ref, acc_ref, cnt_ref):
    i = pl.program_id(0)

    @pl.when(i == 0)
    def _():
        acc_ref[...] = jnp.zeros_like(acc_ref)
        cnt_ref[...] = jnp.zeros_like(cnt_ref)

    num = pn_ref[0] + pn_ref[1]
    den = pd_ref[0, :, 0:1] + pd_ref[1, :, 0:1]
    ws = jnp.exp(as_ref[...] - m_ref[0, 0])
    num = num + ws * xl_ref[...]
    den = den + ws
    o = num / (den + 1e-16) + bias_ref[...]
    o = jnp.maximum(o, 0.0)
    oh = oh_ref[...]
    dims = (((0,), (0,)), ((), ()))
    acc_ref[...] = acc_ref[...] + lax.dot_general(
        oh, o, dims, preferred_element_type=jnp.float32)
    ones = jnp.ones((oh.shape[0], 128), jnp.float32)
    cnt_ref[...] = cnt_ref[...] + lax.dot_general(
        oh, ones, dims, preferred_element_type=jnp.float32)

    @pl.when(i == n_blocks - 1)
    def _():
        pooled = acc_ref[...] / jnp.maximum(cnt_ref[...], 1.0)
        out_ref[...] = jnp.dot(pooled, wc_ref[...],
                               preferred_element_type=jnp.float32) + bc_ref[...]


def _epilogue(pn, pd, aself, xl, onehot, bias_row, m11, Wc, bc_row, n_nodes):
    n_acc = pn.shape[1]
    g = onehot.shape[1]
    c = Wc.shape[1]
    blk = 1000
    n_blocks = n_nodes // blk
    return pl.pallas_call(
        functools.partial(_k5_body, n_blocks),
        grid=(n_blocks,),
        in_specs=[
            pl.BlockSpec((2, blk, 128), lambda i: (0, i, 0)),
            pl.BlockSpec((2, blk, 128), lambda i: (0, i, 0)),
            pl.BlockSpec((blk, 1), lambda i: (i, 0)),
            pl.BlockSpec((blk, 128), lambda i: (i, 0)),
            pl.BlockSpec((blk, g), lambda i: (i, 0)),
            pl.BlockSpec((1, 128), lambda i: (0, 0)),
            pl.BlockSpec((1, 1), lambda i: (0, 0)),
            pl.BlockSpec((128, c), lambda i: (0, 0)),
            pl.BlockSpec((1, c), lambda i: (0, 0)),
        ],
        out_specs=pl.BlockSpec((g, c), lambda i: (0, 0)),
        out_shape=jax.ShapeDtypeStruct((g, c), jnp.float32),
        scratch_shapes=[
            pltpu.VMEM((g, 128), jnp.float32),
            pltpu.VMEM((g, 128), jnp.float32),
        ],
        compiler_params=pltpu.CompilerParams(
            dimension_semantics=("arbitrary",),
        ),
    )(pn, pd, aself, xl, onehot, bias_row, m11, Wc, bc_row)


def kernel(x, edge_index, edge_attr, batch, Wl, Wr, We, att, bias, Wc, bc):
    n = x.shape[0]
    e = edge_index.shape[1]

    src = edge_index[0]
    dst = edge_index[1]

    e_pad = _cdiv(e, NW * CHUNK) * NW * CHUNK
    n_pad = _cdiv(n, NS * CHUNK) * NS * CHUNK
    dump_row = n

    src_pad = jnp.concatenate([src, jnp.zeros((e_pad - e,), jnp.int32)])
    dst_a = jnp.concatenate([dst, jnp.zeros((e_pad - e,), jnp.int32)])
    dst_b = jnp.concatenate(
        [dst, jnp.full((e_pad - e,), dump_row, jnp.int32)])
    ea_pad = jnp.concatenate(
        [edge_attr, jnp.zeros((e_pad - e, edge_attr.shape[1]), jnp.float32)])
    x_pad = jnp.concatenate(
        [x, jnp.zeros((n_pad - n, x.shape[1]), jnp.float32)])

    el, _, sl = _edge_transform(ea_pad, We, e)

    xl, xr, aself = _node_transform(x_pad, Wl, Wr, sl, att.reshape(128, 1))

    araw, tmax = _edge_logits(xl, xr, el, src_pad, dst_a, att)

    m = jnp.maximum(jnp.max(tmax), jnp.max(aself))
    m_vec = jnp.full((LANES,), m, jnp.float32)

    if _DEBUG_JNP_TAIL:
        w = jnp.exp(araw[:e] - m)
        ws = jnp.exp(aself[:n, 0] - m)
        den = jax.ops.segment_sum(w, dst, num_segments=n) + ws
        num = jax.ops.segment_sum(xl[src] * w[:, None], dst, num_segments=n)
        num = num + ws[:, None] * xl[:n]
        o = jnp.maximum(num / (den[:, None] + 1e-16) + bias, 0.0)
        sums = jax.ops.segment_sum(o, batch, num_segments=64)
        counts = jax.ops.segment_sum(jnp.ones((n,), jnp.float32), batch,
                                     num_segments=64)
        pooled = sums / jnp.maximum(counts, 1.0)[:, None]
        return pooled @ Wc + bc

    pn = _scatter_messages(xl, src_pad, dst_b, araw, m_vec, n_pad)
    pd = _scatter_weights(dst_b, araw, m_vec, n_pad)
    pn = pn.reshape(NC, n_pad, 128)
    pd = pd.reshape(NC, n_pad, 128)

    if _DEBUG_JNP_EPILOGUE:
        ws = jnp.exp(aself[:n, 0] - m)
        num = pn[0, :n] + pn[1, :n] + ws[:, None] * xl[:n]
        den = pd[0, :n, 0] + pd[1, :n, 0] + ws
        o = jnp.maximum(num / (den[:, None] + 1e-16) + bias, 0.0)
        sums = jax.ops.segment_sum(o, batch, num_segments=64)
        counts = jax.ops.segment_sum(jnp.ones((n,), jnp.float32), batch,
                                     num_segments=64)
        pooled = sums / jnp.maximum(counts, 1.0)[:, None]
        return pooled @ Wc + bc

    gids = jnp.arange(64, dtype=jnp.int32)
    onehot = (batch[:, None] == gids[None, :]).astype(jnp.float32)
    out = _epilogue(pn, pd, aself, xl, onehot,
                    bias.reshape(1, 128), m.reshape(1, 1),
                    Wc, bc.reshape(1, -1), n)
    return out


# speedup vs baseline: 5.2444x; 1.2106x over previous
import functools

import jax
import jax.numpy as jnp
from jax import lax
from jax.experimental import pallas as pl
from jax.experimental.pallas import tpu as pltpu
from jax.experimental.pallas import tpu_sc as plsc

NC = 2
NS = 16
NW = NC * NS
LANES = 16
CHUNK = 128


def _cdiv(a, b):
    return (a + b - 1) // b


_DEBUG_JNP_TAIL = False
_DEBUG_JNP_EPILOGUE = False


def _k1_body(e_total, n_blocks, ea_ref, we_ref, el_ref, cs_ref, sl_ref):
    i = pl.program_id(0)

    @pl.when(i == 0)
    def _():
        cs_ref[...] = jnp.zeros_like(cs_ref)

    ea = ea_ref[...]
    el_ref[...] = jnp.dot(ea, we_ref[...], preferred_element_type=jnp.float32)
    cs_ref[0, :16] = cs_ref[0, :16] + jnp.sum(ea, axis=0)

    @pl.when(i == n_blocks - 1)
    def _():
        mean = (cs_ref[0, :16] / jnp.float32(e_total)).reshape(1, 16)
        sl_ref[...] = jnp.dot(mean, we_ref[...], preferred_element_type=jnp.float32)


def _edge_transform(ea_pad, We, e_total):
    e_pad = ea_pad.shape[0]
    blk = 4096
    n_blocks = e_pad // blk
    return pl.pallas_call(
        functools.partial(_k1_body, e_total, n_blocks),
        grid=(n_blocks,),
        in_specs=[
            pl.BlockSpec((blk, 16), lambda i: (i, 0)),
            pl.BlockSpec((16, 128), lambda i: (0, 0)),
        ],
        out_specs=[
            pl.BlockSpec((blk, 128), lambda i: (i, 0)),
            pl.BlockSpec((8, 128), lambda i: (0, 0)),
            pl.BlockSpec((1, 128), lambda i: (0, 0)),
        ],
        out_shape=[
            jax.ShapeDtypeStruct((e_pad, 128), jnp.float32),
            jax.ShapeDtypeStruct((8, 128), jnp.float32),
            jax.ShapeDtypeStruct((1, 128), jnp.float32),
        ],
        compiler_params=pltpu.CompilerParams(
            dimension_semantics=("arbitrary",),
        ),
    )(ea_pad, We)


def _k2_body(x_ref, wl_ref, wr_ref, sl_ref, att_ref, xl_ref, xr_ref, as_ref):
    xb = x_ref[...]
    xl = jnp.dot(xb, wl_ref[...], preferred_element_type=jnp.float32)
    xr = jnp.dot(xb, wr_ref[...], preferred_element_type=jnp.float32)
    xl_ref[...] = xl
    xr_ref[...] = xr
    t = xl + xr + sl_ref[...]
    t = jnp.maximum(t, 0.2 * t)
    as_ref[...] = jnp.dot(t, att_ref[...], preferred_element_type=jnp.float32)


def _node_transform(x_pad, Wl, Wr, sl, att_col):
    n_pad = x_pad.shape[0]
    blk = 1280
    n_blocks = n_pad // blk
    return pl.pallas_call(
        _k2_body,
        grid=(n_blocks,),
        in_specs=[
            pl.BlockSpec((blk, 128), lambda i: (i, 0)),
            pl.BlockSpec((128, 128), lambda i: (0, 0)),
            pl.BlockSpec((128, 128), lambda i: (0, 0)),
            pl.BlockSpec((1, 128), lambda i: (0, 0)),
            pl.BlockSpec((128, 1), lambda i: (0, 0)),
        ],
        out_specs=[
            pl.BlockSpec((blk, 128), lambda i: (i, 0)),
            pl.BlockSpec((blk, 128), lambda i: (i, 0)),
            pl.BlockSpec((blk, 1), lambda i: (i, 0)),
        ],
        out_shape=[
            jax.ShapeDtypeStruct((n_pad, 128), jnp.float32),
            jax.ShapeDtypeStruct((n_pad, 128), jnp.float32),
            jax.ShapeDtypeStruct((n_pad, 1), jnp.float32),
        ],
    )(x_pad, Wl, Wr, sl, att_col)


def _butterfly_sum(v, perms):
    for p in perms:
        v = v + v.at[p].get(mode="promise_in_bounds")
    return v


def _k3_body(epw, cha, xl_hbm, xr_hbm, el_hbm, src_hbm, dst_hbm, att_hbm,
             araw_hbm, tmax_hbm,
             s_idx, d_idx, xlb, xrb, elb, ab, attv, mb,
             semx0, semx1, semr0, semr1, seme0, seme1):
    wid = lax.axis_index("s") * NC + lax.axis_index("c")
    base = wid * epw
    n_chunks = epw // cha
    semx = (semx0, semx1)
    semr = (semr0, semr1)
    seme = (seme0, seme1)

    pltpu.sync_copy(att_hbm, attv)
    pltpu.sync_copy(src_hbm.at[pl.ds(base, epw)], s_idx)
    pltpu.sync_copy(dst_hbm.at[pl.ds(base, epw)], d_idx)
    att_g = [attv[pl.ds(g * 16, 16)] for g in range(8)]
    lane = lax.iota(jnp.int32, 16)
    perms = [lane ^ (1 << k) for k in range(4)]

    def issue(c, b):
        pltpu.async_copy(
            xl_hbm.at[s_idx.at[pl.ds(c * cha, cha)]], xlb.at[b], semx[b])
        pltpu.async_copy(
            xr_hbm.at[d_idx.at[pl.ds(c * cha, cha)]], xrb.at[b], semr[b])
        pltpu.async_copy(
            el_hbm.at[pl.ds(base + c * cha, cha)], elb.at[b], seme[b])

    def drain(b):
        pltpu.make_async_copy(
            xl_hbm.at[pl.ds(0, cha)], xlb.at[b], semx[b]).wait()
        pltpu.make_async_copy(
            xr_hbm.at[pl.ds(0, cha)], xrb.at[b], semr[b]).wait()
        pltpu.make_async_copy(
            el_hbm.at[pl.ds(0, cha)], elb.at[b], seme[b]).wait()

    def compute(c, b, rmax):
        def grp_body(j16, rmax_in):
            av = jnp.zeros((16,), jnp.float32)
            for jj in range(16):
                j = j16 * 16 + jj
                acc = jnp.zeros((16,), jnp.float32)
                for g in range(8):
                    v = (xlb[b, j, pl.ds(g * 16, 16)]
                         + xrb[b, j, pl.ds(g * 16, 16)]
                         + elb[b, j, pl.ds(g * 16, 16)])
                    v = jnp.maximum(v, 0.2 * v)
                    acc = acc + v * att_g[g]
                sm = _butterfly_sum(acc, perms)
                av = jnp.where(lane == jj, sm, av)
            ab[pl.ds(c * cha + j16 * 16, 16)] = av
            return jnp.maximum(rmax_in, av)

        return lax.fori_loop(0, cha // 16, grp_body, rmax)

    issue(0, 0)
    issue(1, 1)

    def pair_body(c2, rmax):
        c = c2 * 2
        drain(0)
        rmax = compute(c, 0, rmax)
        issue(c + 2, 0)
        drain(1)
        rmax = compute(c + 1, 1, rmax)
        issue(c + 3, 1)
        return rmax

    rmax = lax.fori_loop(0, n_chunks // 2 - 1, pair_body,
                         jnp.full((16,), -jnp.inf, jnp.float32))
    drain(0)
    rmax = compute(n_chunks - 2, 0, rmax)
    drain(1)
    rmax = compute(n_chunks - 1, 1, rmax)

    pltpu.sync_copy(ab, araw_hbm.at[pl.ds(base, epw)])
    mb[...] = rmax
    pltpu.sync_copy(mb, tmax_hbm.at[wid])


def _edge_logits(xl, xr, el, src, dst, att):
    e_pad = src.shape[0]
    epw = e_pad // NW
    mesh = plsc.VectorSubcoreMesh(core_axis_name="c", subcore_axis_name="s",
                                  num_cores=NC, num_subcores=NS)
    cha = 64
    kfun = pl.kernel(
        functools.partial(_k3_body, epw, cha),
        out_type=[
            jax.ShapeDtypeStruct((e_pad,), jnp.float32),
            jax.ShapeDtypeStruct((NW, LANES), jnp.float32),
        ],
        mesh=mesh,
        scratch_types=[
            pltpu.VMEM((epw,), jnp.int32),
            pltpu.VMEM((epw,), jnp.int32),
            pltpu.VMEM((2, cha, 128), jnp.float32),
            pltpu.VMEM((2, cha, 128), jnp.float32),
            pltpu.VMEM((2, cha, 128), jnp.float32),
            pltpu.VMEM((epw,), jnp.float32),
            pltpu.VMEM((128,), jnp.float32),
            pltpu.VMEM((LANES,), jnp.float32),
            pltpu.SemaphoreType.DMA,
            pltpu.SemaphoreType.DMA,
            pltpu.SemaphoreType.DMA,
            pltpu.SemaphoreType.DMA,
            pltpu.SemaphoreType.DMA,
            pltpu.SemaphoreType.DMA,
        ],
    )
    return kfun(xl, xr, el, src, dst, att)


def _k4_body(epw, n_acc, chb, xl_hbm, src_hbm, dst_hbm, araw_hbm, m_hbm,
             pn_hbm,
             s_idx, d_idx, xlb, ab, rowb, mv,
             shn, sem0):
    cid = lax.axis_index("c")
    sid = lax.axis_index("s")
    wid = sid * NC + cid
    base = wid * epw
    n_chunks = epw // chb
    rows_per_tile = n_acc // NS

    lane = lax.iota(jnp.int32, 16)
    zero16 = jnp.zeros((16,), jnp.float32)

    def zrow(r, _):
        for g in range(8):
            rowb[r, pl.ds(g * 16, 16)] = zero16
        return 0

    lax.fori_loop(0, chb, zrow, 0)

    def zstripe(z, _):
        pltpu.sync_copy(rowb, shn.at[pl.ds(sid * rows_per_tile + z * chb, chb)])
        return 0

    lax.fori_loop(0, rows_per_tile // chb, zstripe, 0)
    plsc.subcore_barrier()

    pltpu.sync_copy(m_hbm, mv)
    mvec = mv[...]

    def chunk_body(c, _):
        off = base + c * chb
        pltpu.sync_copy(src_hbm.at[pl.ds(off, chb)], s_idx)
        pltpu.sync_copy(dst_hbm.at[pl.ds(off, chb)], d_idx)
        cp0 = pltpu.async_copy(xl_hbm.at[s_idx], xlb, sem0)
        pltpu.sync_copy(araw_hbm.at[pl.ds(off, chb)], ab)
        cp0.wait()

        def grp_body(j16, _):
            av = ab[pl.ds(j16 * 16, 16)]
            wv = jnp.exp(av - mvec)
            for jj in range(16):
                j = j16 * 16 + jj
                wb = wv.at[lane * 0 + jj].get(mode="promise_in_bounds")
                for g in range(8):
                    rowb[j, pl.ds(g * 16, 16)] = xlb[j, pl.ds(g * 16, 16)] * wb
            return 0

        lax.fori_loop(0, chb // 16, grp_body, 0)
        pltpu.sync_copy(rowb, shn.at[d_idx], add=True)
        return 0

    lax.fori_loop(0, n_chunks, chunk_body, 0)
    plsc.subcore_barrier()

    r0 = sid * rows_per_tile
    pltpu.sync_copy(shn.at[pl.ds(r0, rows_per_tile)],
                    pn_hbm.at[pl.ds(cid * n_acc + r0, rows_per_tile)])


def _scatter_messages(xl, src, dst, araw, m_vec, n_acc):
    e_pad = src.shape[0]
    epw = e_pad // NW
    chb = 64
    mesh = plsc.VectorSubcoreMesh(core_axis_name="c", subcore_axis_name="s",
                                  num_cores=NC, num_subcores=NS)
    kfun = pl.kernel(
        functools.partial(_k4_body, epw, n_acc, chb),
        out_type=jax.ShapeDtypeStruct((NC * n_acc, 128), jnp.float32),
        mesh=mesh,
        scratch_types=[
            pltpu.VMEM((chb,), jnp.int32),
            pltpu.VMEM((chb,), jnp.int32),
            pltpu.VMEM((chb, 128), jnp.float32),
            pltpu.VMEM((chb,), jnp.float32),
            pltpu.VMEM((chb, 128), jnp.float32),
            pltpu.VMEM((LANES,), jnp.float32),
            pltpu.VMEM_SHARED((n_acc, 128), jnp.float32),
            pltpu.SemaphoreType.DMA,
        ],
    )
    return kfun(xl, src, dst, araw, m_vec)


def _k4b_body(epw, n_acc, chb, dst_hbm, araw_hbm, m_hbm,
              pd_hbm,
              d_idx, ab, wrowb, mv, shd):
    cid = lax.axis_index("c")
    sid = lax.axis_index("s")
    wid = sid * NC + cid
    base = wid * epw
    n_chunks = epw // chb
    rows_per_tile = n_acc // NS

    lane = lax.iota(jnp.int32, 16)
    zero16 = jnp.zeros((16,), jnp.float32)

    def zrow(r, _):
        for g in range(8):
            wrowb[r, pl.ds(g * 16, 16)] = zero16
        return 0

    lax.fori_loop(0, chb, zrow, 0)

    def zstripe(z, _):
        pltpu.sync_copy(wrowb, shd.at[pl.ds(sid * rows_per_tile + z * chb, chb)])
        return 0

    lax.fori_loop(0, rows_per_tile // chb, zstripe, 0)
    plsc.subcore_barrier()

    pltpu.sync_copy(m_hbm, mv)
    mvec = mv[...]

    def chunk_body(c, _):
        off = base + c * chb
        pltpu.sync_copy(dst_hbm.at[pl.ds(off, chb)], d_idx)
        pltpu.sync_copy(araw_hbm.at[pl.ds(off, chb)], ab)

        def grp_body(j16, _):
            av = ab[pl.ds(j16 * 16, 16)]
            wv = jnp.exp(av - mvec)
            for jj in range(16):
                j = j16 * 16 + jj
                wb = wv.at[lane * 0 + jj].get(mode="promise_in_bounds")
                wrowb[j, pl.ds(0, 16)] = jnp.where(lane == 0, wb, zero16)
            return 0

        lax.fori_loop(0, chb // 16, grp_body, 0)
        pltpu.sync_copy(wrowb, shd.at[d_idx], add=True)
        return 0

    lax.fori_loop(0, n_chunks, chunk_body, 0)
    plsc.subcore_barrier()

    r0 = sid * rows_per_tile
    pltpu.sync_copy(shd.at[pl.ds(r0, rows_per_tile)],
                    pd_hbm.at[pl.ds(cid * n_acc + r0, rows_per_tile)])


def _scatter_weights(dst, araw, m_vec, n_acc):
    e_pad = dst.shape[0]
    epw = e_pad // NW
    chb = 64
    mesh = plsc.VectorSubcoreMesh(core_axis_name="c", subcore_axis_name="s",
                                  num_cores=NC, num_subcores=NS)
    kfun = pl.kernel(
        functools.partial(_k4b_body, epw, n_acc, chb),
        out_type=jax.ShapeDtypeStruct((NC * n_acc, 128), jnp.float32),
        mesh=mesh,
        scratch_types=[
            pltpu.VMEM((chb,), jnp.int32),
            pltpu.VMEM((chb,), jnp.float32),
            pltpu.VMEM((chb, 128), jnp.float32),
            pltpu.VMEM((LANES,), jnp.float32),
            pltpu.VMEM_SHARED((n_acc, 128), jnp.float32),
        ],
    )
    return kfun(dst, araw, m_vec)


def _k5_body(n_blocks, pn_ref, pd_ref, as_ref, xl_ref, oh_ref, bias_ref,
             m_ref, wc_ref, bc_ref, out_ref, acc_ref, cnt_ref):
    i = pl.program_id(0)

    @pl.when(i == 0)
    def _():
        acc_ref[...] = jnp.zeros_like(acc_ref)
        cnt_ref[...] = jnp.zeros_like(cnt_ref)

    num = pn_ref[0] + pn_ref[1]
    den = pd_ref[0, :, 0:1] + pd_ref[1, :, 0:1]
    ws = jnp.exp(as_ref[...] - m_ref[0, 0])
    num = num + ws * xl_ref[...]
    den = den + ws
    o = num / (den + 1e-16) + bias_ref[...]
    o = jnp.maximum(o, 0.0)
    oh = oh_ref[...]
    dims = (((0,), (0,)), ((), ()))
    acc_ref[...] = acc_ref[...] + lax.dot_general(
        oh, o, dims, preferred_element_type=jnp.float32)
    ones = jnp.ones((oh.shape[0], 128), jnp.float32)
    cnt_ref[...] = cnt_ref[...] + lax.dot_general(
        oh, ones, dims, preferred_element_type=jnp.float32)

    @pl.when(i == n_blocks - 1)
    def _():
        pooled = acc_ref[...] / jnp.maximum(cnt_ref[...], 1.0)
        out_ref[...] = jnp.dot(pooled, wc_ref[...],
                               preferred_element_type=jnp.float32) + bc_ref[...]


def _epilogue(pn, pd, aself, xl, onehot, bias_row, m11, Wc, bc_row, n_nodes):
    n_acc = pn.shape[1]
    g = onehot.shape[1]
    c = Wc.shape[1]
    blk = 1000
    n_blocks = n_nodes // blk
    return pl.pallas_call(
        functools.partial(_k5_body, n_blocks),
        grid=(n_blocks,),
        in_specs=[
            pl.BlockSpec((2, blk, 128), lambda i: (0, i, 0)),
            pl.BlockSpec((2, blk, 128), lambda i: (0, i, 0)),
            pl.BlockSpec((blk, 1), lambda i: (i, 0)),
            pl.BlockSpec((blk, 128), lambda i: (i, 0)),
            pl.BlockSpec((blk, g), lambda i: (i, 0)),
            pl.BlockSpec((1, 128), lambda i: (0, 0)),
            pl.BlockSpec((1, 1), lambda i: (0, 0)),
            pl.BlockSpec((128, c), lambda i: (0, 0)),
            pl.BlockSpec((1, c), lambda i: (0, 0)),
        ],
        out_specs=pl.BlockSpec((g, c), lambda i: (0, 0)),
        out_shape=jax.ShapeDtypeStruct((g, c), jnp.float32),
        scratch_shapes=[
            pltpu.VMEM((g, 128), jnp.float32),
            pltpu.VMEM((g, 128), jnp.float32),
        ],
        compiler_params=pltpu.CompilerParams(
            dimension_semantics=("arbitrary",),
        ),
    )(pn, pd, aself, xl, onehot, bias_row, m11, Wc, bc_row)


def kernel(x, edge_index, edge_attr, batch, Wl, Wr, We, att, bias, Wc, bc):
    n = x.shape[0]
    e = edge_index.shape[1]

    src = edge_index[0]
    dst = edge_index[1]

    e_pad = _cdiv(e, NW * CHUNK) * NW * CHUNK
    n_pad = _cdiv(n, NS * CHUNK) * NS * CHUNK
    dump_row = n

    src_pad = jnp.concatenate([src, jnp.zeros((e_pad - e,), jnp.int32)])
    dst_a = jnp.concatenate([dst, jnp.zeros((e_pad - e,), jnp.int32)])
    dst_b = jnp.concatenate(
        [dst, jnp.full((e_pad - e,), dump_row, jnp.int32)])
    ea_pad = jnp.concatenate(
        [edge_attr, jnp.zeros((e_pad - e, edge_attr.shape[1]), jnp.float32)])
    x_pad = jnp.concatenate(
        [x, jnp.zeros((n_pad - n, x.shape[1]), jnp.float32)])

    el, _, sl = _edge_transform(ea_pad, We, e)

    xl, xr, aself = _node_transform(x_pad, Wl, Wr, sl, att.reshape(128, 1))

    araw, tmax = _edge_logits(xl, xr, el, src_pad, dst_a, att)

    m = jnp.maximum(jnp.max(tmax), jnp.max(aself))
    m_vec = jnp.full((LANES,), m, jnp.float32)

    if _DEBUG_JNP_TAIL:
        w = jnp.exp(araw[:e] - m)
        ws = jnp.exp(aself[:n, 0] - m)
        den = jax.ops.segment_sum(w, dst, num_segments=n) + ws
        num = jax.ops.segment_sum(xl[src] * w[:, None], dst, num_segments=n)
        num = num + ws[:, None] * xl[:n]
        o = jnp.maximum(num / (den[:, None] + 1e-16) + bias, 0.0)
        sums = jax.ops.segment_sum(o, batch, num_segments=64)
        counts = jax.ops.segment_sum(jnp.ones((n,), jnp.float32), batch,
                                     num_segments=64)
        pooled = sums / jnp.maximum(counts, 1.0)[:, None]
        return pooled @ Wc + bc

    pn = _scatter_messages(xl, src_pad, dst_b, araw, m_vec, n_pad)
    pd = _scatter_weights(dst_b, araw, m_vec, n_pad)
    pn = pn.reshape(NC, n_pad, 128)
    pd = pd.reshape(NC, n_pad, 128)

    if _DEBUG_JNP_EPILOGUE:
        ws = jnp.exp(aself[:n, 0] - m)
        num = pn[0, :n] + pn[1, :n] + ws[:, None] * xl[:n]
        den = pd[0, :n, 0] + pd[1, :n, 0] + ws
        o = jnp.maximum(num / (den[:, None] + 1e-16) + bias, 0.0)
        sums = jax.ops.segment_sum(o, batch, num_segments=64)
        counts = jax.ops.segment_sum(jnp.ones((n,), jnp.float32), batch,
                                     num_segments=64)
        pooled = sums / jnp.maximum(counts, 1.0)[:, None]
        return pooled @ Wc + bc

    gids = jnp.arange(64, dtype=jnp.int32)
    onehot = (batch[:, None] == gids[None, :]).astype(jnp.float32)
    out = _epilogue(pn, pd, aself, xl, onehot,
                    bias.reshape(1, 128), m.reshape(1, 1),
                    Wc, bc.reshape(1, -1), n)
    return out


# speedup vs baseline: 6.0630x; 1.1561x over previous
import functools

import jax
import jax.numpy as jnp
from jax import lax
from jax.experimental import pallas as pl
from jax.experimental.pallas import tpu as pltpu
from jax.experimental.pallas import tpu_sc as plsc

NC = 2
NS = 16
NW = NC * NS
LANES = 16
CHUNK = 128


def _cdiv(a, b):
    return (a + b - 1) // b


_DEBUG_JNP_TAIL = False
_DEBUG_JNP_EPILOGUE = False


def _k1_body(e_total, n_blocks, ea_ref, we_ref, el_ref, cs_ref, sl_ref):
    i = pl.program_id(0)

    @pl.when(i == 0)
    def _():
        cs_ref[...] = jnp.zeros_like(cs_ref)

    ea = ea_ref[...]
    el_ref[...] = jnp.dot(ea, we_ref[...], preferred_element_type=jnp.float32)
    cs_ref[0, :16] = cs_ref[0, :16] + jnp.sum(ea, axis=0)

    @pl.when(i == n_blocks - 1)
    def _():
        mean = (cs_ref[0, :16] / jnp.float32(e_total)).reshape(1, 16)
        sl_ref[...] = jnp.dot(mean, we_ref[...], preferred_element_type=jnp.float32)


def _edge_transform(ea_pad, We, e_total):
    e_pad = ea_pad.shape[0]
    blk = 4096
    n_blocks = e_pad // blk
    return pl.pallas_call(
        functools.partial(_k1_body, e_total, n_blocks),
        grid=(n_blocks,),
        in_specs=[
            pl.BlockSpec((blk, 16), lambda i: (i, 0)),
            pl.BlockSpec((16, 128), lambda i: (0, 0)),
        ],
        out_specs=[
            pl.BlockSpec((blk, 128), lambda i: (i, 0)),
            pl.BlockSpec((8, 128), lambda i: (0, 0)),
            pl.BlockSpec((1, 128), lambda i: (0, 0)),
        ],
        out_shape=[
            jax.ShapeDtypeStruct((e_pad, 128), jnp.float32),
            jax.ShapeDtypeStruct((8, 128), jnp.float32),
            jax.ShapeDtypeStruct((1, 128), jnp.float32),
        ],
        compiler_params=pltpu.CompilerParams(
            dimension_semantics=("arbitrary",),
        ),
    )(ea_pad, We)


def _k2_body(x_ref, wl_ref, wr_ref, sl_ref, att_ref, xl_ref, xr_ref, as_ref):
    xb = x_ref[...]
    xl = jnp.dot(xb, wl_ref[...], preferred_element_type=jnp.float32)
    xr = jnp.dot(xb, wr_ref[...], preferred_element_type=jnp.float32)
    xl_ref[...] = xl
    xr_ref[...] = xr
    t = xl + xr + sl_ref[...]
    t = jnp.maximum(t, 0.2 * t)
    as_ref[...] = jnp.dot(t, att_ref[...], preferred_element_type=jnp.float32)


def _node_transform(x_pad, Wl, Wr, sl, att_col):
    n_pad = x_pad.shape[0]
    blk = 1280
    n_blocks = n_pad // blk
    return pl.pallas_call(
        _k2_body,
        grid=(n_blocks,),
        in_specs=[
            pl.BlockSpec((blk, 128), lambda i: (i, 0)),
            pl.BlockSpec((128, 128), lambda i: (0, 0)),
            pl.BlockSpec((128, 128), lambda i: (0, 0)),
            pl.BlockSpec((1, 128), lambda i: (0, 0)),
            pl.BlockSpec((128, 1), lambda i: (0, 0)),
        ],
        out_specs=[
            pl.BlockSpec((blk, 128), lambda i: (i, 0)),
            pl.BlockSpec((blk, 128), lambda i: (i, 0)),
            pl.BlockSpec((blk, 1), lambda i: (i, 0)),
        ],
        out_shape=[
            jax.ShapeDtypeStruct((n_pad, 128), jnp.float32),
            jax.ShapeDtypeStruct((n_pad, 128), jnp.float32),
            jax.ShapeDtypeStruct((n_pad, 1), jnp.float32),
        ],
    )(x_pad, Wl, Wr, sl, att_col)


def _butterfly_sum(v, perms):
    for p in perms:
        v = v + v.at[p].get(mode="promise_in_bounds")
    return v


def _k3_body(epw, cha, xl_hbm, xr_hbm, el_hbm, src_hbm, dst_hbm, att_hbm,
             araw_hbm, tmax_hbm,
             s_idx, d_idx, xlb, xrb, elb, ab, attv, mb,
             semx0, semx1, semr0, semr1, seme0, seme1):
    wid = lax.axis_index("s") * NC + lax.axis_index("c")
    base = wid * epw
    n_chunks = epw // cha
    semx = (semx0, semx1)
    semr = (semr0, semr1)
    seme = (seme0, seme1)

    pltpu.sync_copy(att_hbm, attv)
    pltpu.sync_copy(src_hbm.at[pl.ds(base, epw)], s_idx)
    pltpu.sync_copy(dst_hbm.at[pl.ds(base, epw)], d_idx)
    att_g = [attv[pl.ds(g * 16, 16)] for g in range(8)]
    lane = lax.iota(jnp.int32, 16)
    perms = [lane ^ (1 << k) for k in range(4)]

    def issue(c, b):
        pltpu.async_copy(
            xl_hbm.at[s_idx.at[pl.ds(c * cha, cha)]], xlb.at[b], semx[b])
        pltpu.async_copy(
            xr_hbm.at[d_idx.at[pl.ds(c * cha, cha)]], xrb.at[b], semr[b])
        pltpu.async_copy(
            el_hbm.at[pl.ds(base + c * cha, cha)], elb.at[b], seme[b])

    def drain(b):
        pltpu.make_async_copy(
            xl_hbm.at[pl.ds(0, cha)], xlb.at[b], semx[b]).wait()
        pltpu.make_async_copy(
            xr_hbm.at[pl.ds(0, cha)], xrb.at[b], semr[b]).wait()
        pltpu.make_async_copy(
            el_hbm.at[pl.ds(0, cha)], elb.at[b], seme[b]).wait()

    def compute(c, b, rmax):
        def grp_body(j16, rmax_in):
            av = jnp.zeros((16,), jnp.float32)
            for jj in range(16):
                j = j16 * 16 + jj
                acc = jnp.zeros((16,), jnp.float32)
                for g in range(8):
                    v = (xlb[b, j, pl.ds(g * 16, 16)]
                         + xrb[b, j, pl.ds(g * 16, 16)]
                         + elb[b, j, pl.ds(g * 16, 16)])
                    v = jnp.maximum(v, 0.2 * v)
                    acc = acc + v * att_g[g]
                sm = _butterfly_sum(acc, perms)
                av = jnp.where(lane == jj, sm, av)
            ab[pl.ds(c * cha + j16 * 16, 16)] = av
            return jnp.maximum(rmax_in, av)

        return lax.fori_loop(0, cha // 16, grp_body, rmax)

    issue(0, 0)
    issue(1, 1)

    def pair_body(c2, rmax):
        c = c2 * 2
        drain(0)
        rmax = compute(c, 0, rmax)
        issue(c + 2, 0)
        drain(1)
        rmax = compute(c + 1, 1, rmax)
        issue(c + 3, 1)
        return rmax

    rmax = lax.fori_loop(0, n_chunks // 2 - 1, pair_body,
                         jnp.full((16,), -jnp.inf, jnp.float32))
    drain(0)
    rmax = compute(n_chunks - 2, 0, rmax)
    drain(1)
    rmax = compute(n_chunks - 1, 1, rmax)

    pltpu.sync_copy(ab, araw_hbm.at[pl.ds(base, epw)])
    mb[...] = rmax
    pltpu.sync_copy(mb, tmax_hbm.at[wid])


def _edge_logits(xl, xr, el, src, dst, att):
    e_pad = src.shape[0]
    epw = e_pad // NW
    mesh = plsc.VectorSubcoreMesh(core_axis_name="c", subcore_axis_name="s",
                                  num_cores=NC, num_subcores=NS)
    cha = 64
    kfun = pl.kernel(
        functools.partial(_k3_body, epw, cha),
        out_type=[
            jax.ShapeDtypeStruct((e_pad,), jnp.float32),
            jax.ShapeDtypeStruct((NW, LANES), jnp.float32),
        ],
        mesh=mesh,
        scratch_types=[
            pltpu.VMEM((epw,), jnp.int32),
            pltpu.VMEM((epw,), jnp.int32),
            pltpu.VMEM((2, cha, 128), jnp.float32),
            pltpu.VMEM((2, cha, 128), jnp.float32),
            pltpu.VMEM((2, cha, 128), jnp.float32),
            pltpu.VMEM((epw,), jnp.float32),
            pltpu.VMEM((128,), jnp.float32),
            pltpu.VMEM((LANES,), jnp.float32),
            pltpu.SemaphoreType.DMA,
            pltpu.SemaphoreType.DMA,
            pltpu.SemaphoreType.DMA,
            pltpu.SemaphoreType.DMA,
            pltpu.SemaphoreType.DMA,
            pltpu.SemaphoreType.DMA,
        ],
    )
    return kfun(xl, xr, el, src, dst, att)


def _k4_body(epw, n_acc, chb, xl_hbm, src_hbm, dst_hbm, araw_hbm, m_hbm,
             pn_hbm,
             s_idx, d_idx, xlb, ab, rowb, mv, shn,
             semg0, semg1, sems0, sems1, semd0, semd1, sema0, sema1,
             semw0, semw1):
    cid = lax.axis_index("c")
    sid = lax.axis_index("s")
    wid = sid * NC + cid
    base = wid * epw
    n_chunks = epw // chb
    n2 = n_chunks // 2
    rows_per_tile = n_acc // NS
    semg = (semg0, semg1)
    sems = (sems0, sems1)
    semd = (semd0, semd1)
    sema = (sema0, sema1)
    semw = (semw0, semw1)
    RING = 8

    lane = lax.iota(jnp.int32, 16)
    zero16 = jnp.zeros((16,), jnp.float32)

    def zrow(r, _):
        for b in range(2):
            for g in range(8):
                rowb[b, r, pl.ds(g * 16, 16)] = zero16
        return 0

    lax.fori_loop(0, chb, zrow, 0)

    def zstripe(z, _):
        pltpu.sync_copy(rowb.at[0],
                        shn.at[pl.ds(sid * rows_per_tile + z * chb, chb)])
        return 0

    lax.fori_loop(0, rows_per_tile // chb, zstripe, 0)
    plsc.subcore_barrier()

    pltpu.sync_copy(m_hbm, mv)
    mvec = mv[...]

    def issue_lin(cc, b):
        off = base + cc * chb
        sl = cc % RING
        pltpu.async_copy(src_hbm.at[pl.ds(off, chb)], s_idx.at[sl], sems[b])
        pltpu.async_copy(dst_hbm.at[pl.ds(off, chb)], d_idx.at[sl], semd[b])
        pltpu.async_copy(araw_hbm.at[pl.ds(off, chb)], ab.at[sl], sema[b])

    def drain_lin(b):
        pltpu.make_async_copy(src_hbm.at[pl.ds(0, chb)], s_idx.at[0],
                              sems[b]).wait()
        pltpu.make_async_copy(dst_hbm.at[pl.ds(0, chb)], d_idx.at[0],
                              semd[b]).wait()
        pltpu.make_async_copy(araw_hbm.at[pl.ds(0, chb)], ab.at[0],
                              sema[b]).wait()

    def issue_gather(cc, b):
        pltpu.async_copy(xl_hbm.at[s_idx.at[cc % RING]], xlb.at[b], semg[b])

    def drain_gather(b):
        pltpu.make_async_copy(xl_hbm.at[pl.ds(0, chb)], xlb.at[b],
                              semg[b]).wait()

    def drain_scatter(b):
        pltpu.make_async_copy(rowb.at[b], shn.at[pl.ds(0, chb)],
                              semw[b]).wait()

    def compute(cc, b):
        sl = cc % RING

        def grp_body(j16, _):
            av = ab[sl, pl.ds(j16 * 16, 16)]
            wv = jnp.exp(av - mvec)
            for jj in range(16):
                j = j16 * 16 + jj
                wb = wv.at[lane * 0 + jj].get(mode="promise_in_bounds")
                for g in range(8):
                    rowb[b, j, pl.ds(g * 16, 16)] = (
                        xlb[b, j, pl.ds(g * 16, 16)] * wb)
            return 0

        lax.fori_loop(0, chb // 16, grp_body, 0)

    def issue_scatter(cc, b):
        pltpu.async_copy(rowb.at[b], shn.at[d_idx.at[cc % RING]], semw[b],
                         add=True)

    issue_lin(0, 0)
    issue_lin(1, 1)
    drain_lin(0)
    issue_gather(0, 0)
    drain_lin(1)
    issue_gather(1, 1)
    issue_lin(2, 0)
    issue_lin(3, 1)

    def pair_body(c2, _):
        c = c2 * 2
        for b in range(2):
            drain_gather(b)

            @pl.when(c2 > 0)
            def _():
                drain_scatter(b)

            compute(c + b, b)
            issue_scatter(c + b, b)
        for b in range(2):
            drain_lin(b)
            issue_gather(c + 2 + b, b)
            pl.when(c2 < n2 - 2)(lambda b=b: issue_lin(c + 4 + b, b))
        return 0

    lax.fori_loop(0, n2 - 1, pair_body, 0)
    c_last = n_chunks - 2
    for b in range(2):
        drain_gather(b)
        drain_scatter(b)
        compute(c_last + b, b)
        issue_scatter(c_last + b, b)
    drain_scatter(0)
    drain_scatter(1)
    plsc.subcore_barrier()

    r0 = sid * rows_per_tile
    pltpu.sync_copy(shn.at[pl.ds(r0, rows_per_tile)],
                    pn_hbm.at[pl.ds(cid * n_acc + r0, rows_per_tile)])


def _scatter_messages(xl, src, dst, araw, m_vec, n_acc):
    e_pad = src.shape[0]
    epw = e_pad // NW
    chb = 32
    mesh = plsc.VectorSubcoreMesh(core_axis_name="c", subcore_axis_name="s",
                                  num_cores=NC, num_subcores=NS)
    kfun = pl.kernel(
        functools.partial(_k4_body, epw, n_acc, chb),
        out_type=jax.ShapeDtypeStruct((NC * n_acc, 128), jnp.float32),
        mesh=mesh,
        scratch_types=[
            pltpu.VMEM((8, chb), jnp.int32),
            pltpu.VMEM((8, chb), jnp.int32),
            pltpu.VMEM((2, chb, 128), jnp.float32),
            pltpu.VMEM((8, chb), jnp.float32),
            pltpu.VMEM((2, chb, 128), jnp.float32),
            pltpu.VMEM((LANES,), jnp.float32),
            pltpu.VMEM_SHARED((n_acc, 128), jnp.float32),
        ] + [pltpu.SemaphoreType.DMA] * 10,
    )
    return kfun(xl, src, dst, araw, m_vec)


def _k4b_body(epw, n_acc, chb, dst_hbm, araw_hbm, m_hbm,
              pd_hbm,
              d_idx, ab, wrowb, mv, shd):
    cid = lax.axis_index("c")
    sid = lax.axis_index("s")
    wid = sid * NC + cid
    base = wid * epw
    n_chunks = epw // chb
    rows_per_tile = n_acc // NS

    lane = lax.iota(jnp.int32, 16)
    zero16 = jnp.zeros((16,), jnp.float32)

    def zrow(r, _):
        for g in range(8):
            wrowb[r, pl.ds(g * 16, 16)] = zero16
        return 0

    lax.fori_loop(0, chb, zrow, 0)

    def zstripe(z, _):
        pltpu.sync_copy(wrowb, shd.at[pl.ds(sid * rows_per_tile + z * chb, chb)])
        return 0

    lax.fori_loop(0, rows_per_tile // chb, zstripe, 0)
    plsc.subcore_barrier()

    pltpu.sync_copy(m_hbm, mv)
    mvec = mv[...]

    def chunk_body(c, _):
        off = base + c * chb
        pltpu.sync_copy(dst_hbm.at[pl.ds(off, chb)], d_idx)
        pltpu.sync_copy(araw_hbm.at[pl.ds(off, chb)], ab)

        def grp_body(j16, _):
            av = ab[pl.ds(j16 * 16, 16)]
            wv = jnp.exp(av - mvec)
            for jj in range(16):
                j = j16 * 16 + jj
                wb = wv.at[lane * 0 + jj].get(mode="promise_in_bounds")
                wrowb[j, pl.ds(0, 16)] = jnp.where(lane == 0, wb, zero16)
            return 0

        lax.fori_loop(0, chb // 16, grp_body, 0)
        pltpu.sync_copy(wrowb, shd.at[d_idx], add=True)
        return 0

    lax.fori_loop(0, n_chunks, chunk_body, 0)
    plsc.subcore_barrier()

    r0 = sid * rows_per_tile
    pltpu.sync_copy(shd.at[pl.ds(r0, rows_per_tile)],
                    pd_hbm.at[pl.ds(cid * n_acc + r0, rows_per_tile)])


def _scatter_weights(dst, araw, m_vec, n_acc):
    e_pad = dst.shape[0]
    epw = e_pad // NW
    chb = 64
    mesh = plsc.VectorSubcoreMesh(core_axis_name="c", subcore_axis_name="s",
                                  num_cores=NC, num_subcores=NS)
    kfun = pl.kernel(
        functools.partial(_k4b_body, epw, n_acc, chb),
        out_type=jax.ShapeDtypeStruct((NC * n_acc, 128), jnp.float32),
        mesh=mesh,
        scratch_types=[
            pltpu.VMEM((chb,), jnp.int32),
            pltpu.VMEM((chb,), jnp.float32),
            pltpu.VMEM((chb, 128), jnp.float32),
            pltpu.VMEM((LANES,), jnp.float32),
            pltpu.VMEM_SHARED((n_acc, 128), jnp.float32),
        ],
    )
    return kfun(dst, araw, m_vec)


def _k5_body(n_blocks, pn_ref, pd_ref, as_ref, xl_ref, oh_ref, bias_ref,
             m_ref, wc_ref, bc_ref, out_ref, acc_ref, cnt_ref):
    i = pl.program_id(0)

    @pl.when(i == 0)
    def _():
        acc_ref[...] = jnp.zeros_like(acc_ref)
        cnt_ref[...] = jnp.zeros_like(cnt_ref)

    num = pn_ref[0] + pn_ref[1]
    den = pd_ref[0, :, 0:1] + pd_ref[1, :, 0:1]
    ws = jnp.exp(as_ref[...] - m_ref[0, 0])
    num = num + ws * xl_ref[...]
    den = den + ws
    o = num / (den + 1e-16) + bias_ref[...]
    o = jnp.maximum(o, 0.0)
    oh = oh_ref[...]
    dims = (((0,), (0,)), ((), ()))
    acc_ref[...] = acc_ref[...] + lax.dot_general(
        oh, o, dims, preferred_element_type=jnp.float32)
    ones = jnp.ones((oh.shape[0], 128), jnp.float32)
    cnt_ref[...] = cnt_ref[...] + lax.dot_general(
        oh, ones, dims, preferred_element_type=jnp.float32)

    @pl.when(i == n_blocks - 1)
    def _():
        pooled = acc_ref[...] / jnp.maximum(cnt_ref[...], 1.0)
        out_ref[...] = jnp.dot(pooled, wc_ref[...],
                               preferred_element_type=jnp.float32) + bc_ref[...]


def _epilogue(pn, pd, aself, xl, onehot, bias_row, m11, Wc, bc_row, n_nodes):
    n_acc = pn.shape[1]
    g = onehot.shape[1]
    c = Wc.shape[1]
    blk = 1000
    n_blocks = n_nodes // blk
    return pl.pallas_call(
        functools.partial(_k5_body, n_blocks),
        grid=(n_blocks,),
        in_specs=[
            pl.BlockSpec((2, blk, 128), lambda i: (0, i, 0)),
            pl.BlockSpec((2, blk, 128), lambda i: (0, i, 0)),
            pl.BlockSpec((blk, 1), lambda i: (i, 0)),
            pl.BlockSpec((blk, 128), lambda i: (i, 0)),
            pl.BlockSpec((blk, g), lambda i: (i, 0)),
            pl.BlockSpec((1, 128), lambda i: (0, 0)),
            pl.BlockSpec((1, 1), lambda i: (0, 0)),
            pl.BlockSpec((128, c), lambda i: (0, 0)),
            pl.BlockSpec((1, c), lambda i: (0, 0)),
        ],
        out_specs=pl.BlockSpec((g, c), lambda i: (0, 0)),
        out_shape=jax.ShapeDtypeStruct((g, c), jnp.float32),
        scratch_shapes=[
            pltpu.VMEM((g, 128), jnp.float32),
            pltpu.VMEM((g, 128), jnp.float32),
        ],
        compiler_params=pltpu.CompilerParams(
            dimension_semantics=("arbitrary",),
        ),
    )(pn, pd, aself, xl, onehot, bias_row, m11, Wc, bc_row)


def kernel(x, edge_index, edge_attr, batch, Wl, Wr, We, att, bias, Wc, bc):
    n = x.shape[0]
    e = edge_index.shape[1]

    src = edge_index[0]
    dst = edge_index[1]

    e_pad = _cdiv(e, NW * CHUNK) * NW * CHUNK
    n_pad = _cdiv(n, NS * CHUNK) * NS * CHUNK
    dump_row = n

    src_pad = jnp.concatenate([src, jnp.zeros((e_pad - e,), jnp.int32)])
    dst_a = jnp.concatenate([dst, jnp.zeros((e_pad - e,), jnp.int32)])
    dst_b = jnp.concatenate(
        [dst, jnp.full((e_pad - e,), dump_row, jnp.int32)])
    ea_pad = jnp.concatenate(
        [edge_attr, jnp.zeros((e_pad - e, edge_attr.shape[1]), jnp.float32)])
    x_pad = jnp.concatenate(
        [x, jnp.zeros((n_pad - n, x.shape[1]), jnp.float32)])

    el, _, sl = _edge_transform(ea_pad, We, e)

    xl, xr, aself = _node_transform(x_pad, Wl, Wr, sl, att.reshape(128, 1))

    araw, tmax = _edge_logits(xl, xr, el, src_pad, dst_a, att)

    m = jnp.maximum(jnp.max(tmax), jnp.max(aself))
    m_vec = jnp.full((LANES,), m, jnp.float32)

    if _DEBUG_JNP_TAIL:
        w = jnp.exp(araw[:e] - m)
        ws = jnp.exp(aself[:n, 0] - m)
        den = jax.ops.segment_sum(w, dst, num_segments=n) + ws
        num = jax.ops.segment_sum(xl[src] * w[:, None], dst, num_segments=n)
        num = num + ws[:, None] * xl[:n]
        o = jnp.maximum(num / (den[:, None] + 1e-16) + bias, 0.0)
        sums = jax.ops.segment_sum(o, batch, num_segments=64)
        counts = jax.ops.segment_sum(jnp.ones((n,), jnp.float32), batch,
                                     num_segments=64)
        pooled = sums / jnp.maximum(counts, 1.0)[:, None]
        return pooled @ Wc + bc

    pn = _scatter_messages(xl, src_pad, dst_b, araw, m_vec, n_pad)
    pd = _scatter_weights(dst_b, araw, m_vec, n_pad)
    pn = pn.reshape(NC, n_pad, 128)
    pd = pd.reshape(NC, n_pad, 128)

    if _DEBUG_JNP_EPILOGUE:
        ws = jnp.exp(aself[:n, 0] - m)
        num = pn[0, :n] + pn[1, :n] + ws[:, None] * xl[:n]
        den = pd[0, :n, 0] + pd[1, :n, 0] + ws
        o = jnp.maximum(num / (den[:, None] + 1e-16) + bias, 0.0)
        sums = jax.ops.segment_sum(o, batch, num_segments=64)
        counts = jax.ops.segment_sum(jnp.ones((n,), jnp.float32), batch,
                                     num_segments=64)
        pooled = sums / jnp.maximum(counts, 1.0)[:, None]
        return pooled @ Wc + bc

    gids = jnp.arange(64, dtype=jnp.int32)
    onehot = (batch[:, None] == gids[None, :]).astype(jnp.float32)
    out = _epilogue(pn, pd, aself, xl, onehot,
                    bias.reshape(1, 128), m.reshape(1, 1),
                    Wc, bc.reshape(1, -1), n)
    return out
